# baseline (speedup 1.0000x reference)
import functools
import math

import jax
import jax.numpy as jnp
from jax import lax
from jax.experimental import pallas as pl
from jax.experimental.pallas import tpu as pltpu
from jax.experimental.pallas import tpu_sc as plsc

N_ITEMS = 10000
N_PERSONS = 100000
N_GRADES = 5
N_RESP = 1000000
N_LEVELS = 10

_NC, _NS, _L = 2, 16, 16
_NW = _NC * _NS
_NPAD = 1 << 20
_W = _NPAD // _NW
_C = 4096
_NCHUNK = _W // _C

_LOG2PI = math.log(2.0 * math.pi)
_LN2 = math.log(2.0)


def _softplus(x):
    return jnp.maximum(x, 0.0) + jnp.log(1.0 + jnp.exp(-jnp.abs(x)))


def _tables_body(a_ref, bb_ref, d0_ref, d1_ref, d2_ref,
                 aw_ref, c0_ref, c1_ref, c2_ref, c3_ref):
    aw_ref[...] = _softplus(a_ref[...])
    c0 = bb_ref[...]
    c1 = c0 + _softplus(d0_ref[...])
    c2 = c1 + _softplus(d1_ref[...])
    c3 = c2 + _softplus(d2_ref[...])
    c0_ref[...] = c0
    c1_ref[...] = c1
    c2_ref[...] = c2
    c3_ref[...] = c3


def _make_tables(aw, bbw, d0, d1, d2):
    w = jax.ShapeDtypeStruct((80, 125), jnp.float32)
    return pl.pallas_call(
        _tables_body, out_shape=[w, w, w, w, w],
    )(aw, bbw, d0, d1, d2)


def _ln16(p):
    bits = plsc.bitcast(p, jnp.int32)
    e = lax.shift_right_logical(bits, 23) - 127
    m = plsc.bitcast((bits & 0x007FFFFF) | 0x3F800000, jnp.float32)
    s = (m - 1.0) / (m + 1.0)
    s2 = s * s
    atanh = s * (1.0 + s2 * (1.0 / 3.0 + s2 * (0.2 + s2 * (1.0 / 7.0))))
    return e.astype(jnp.float32) * _LN2 + 2.0 * atanh


def _sc_body(a_hbm, b6t_hbm, t_hbm, idx_hbm, out_hbm,
             a_v, b6_v, t_sh, acc_v,
             x0, x1, pn0, pn1, t0, t1,
             si0, si1, st0, st1, so):
    cid = lax.axis_index("c")
    sid = lax.axis_index("s")
    wid = sid * _NC + cid
    base = wid * _W

    @pl.when(sid == 0)
    def _():
        pltpu.sync_copy(t_hbm, t_sh)

    pltpu.sync_copy(a_hbm, a_v)
    pltpu.sync_copy(b6t_hbm, b6_v)
    plsc.subcore_barrier()

    xb = (x0, x1)
    pb = (pn0, pn1)
    tb = (t0, t1)
    isem = (si0, si1)
    tsem = (st0, st1)
    descs = {}
    lane = lax.iota(jnp.int32, _L)
    lane3 = lane * 3

    def off_of(g):
        return jnp.minimum(base + g * _C, N_RESP - _C)

    def fire_idx(g):
        b = g % 2
        descs[("i", g)] = pltpu.async_copy(
            idx_hbm.at[pl.ds(off_of(g) * 3, 3 * _C)], xb[b], isem[b])

    def unpack(g):
        b = g % 2

        @plsc.parallel_loop(0, _C, step=_L, unroll=4)
        def _(i):
            pb[b][pl.ds(i, _L)] = plsc.load_gather(xb[b], [lane3 + 3 * i + 1])

    def fire_t(g):
        b = g % 2
        descs[("t", g)] = pltpu.async_copy(t_sh.at[pb[b]], tb[b], tsem[b])

    def compute(g, acc_in):
        b = g % 2
        off_c = off_of(g)
        off_own = base + g * _C

        @plsc.parallel_loop(0, _C, step=_L, unroll=4, carry=acc_in)
        def acc_out(i, acc):
            i3 = lane3 + 3 * i
            it = plsc.load_gather(xb[b], [i3])
            rs = plsc.load_gather(xb[b], [i3 + 2])
            mu = rs * N_ITEMS + it
            bu = plsc.load_gather(b6_v, [mu])
            bl = plsc.load_gather(b6_v, [mu - N_ITEMS])
            a16 = plsc.load_gather(a_v, [it])
            tt = tb[b][pl.ds(i, _L)]
            zl = jnp.maximum(a16 * (tt - bl), -30.0)
            zu = jnp.maximum(a16 * (tt - bu), -30.0)
            x = jnp.exp(-zl)
            y = jnp.exp(-zu)
            p = (y - x) / ((1.0 + x) * (1.0 + y))
            lnp = _ln16(jnp.maximum(p, 1e-37))
            pos = off_c + i + lane
            ok = (pos >= off_own) & (pos < N_RESP)
            return acc + jnp.where(ok, lnp, 0.0)

        return acc_out

    fire_idx(0)
    descs[("i", 0)].wait()
    unpack(0)
    fire_t(0)
    if _NCHUNK > 1:
        fire_idx(1)
    acc = jnp.zeros((_L,), jnp.float32)
    for g in range(_NCHUNK):
        if g + 1 < _NCHUNK:
            descs[("i", g + 1)].wait()
            unpack(g + 1)
            fire_t(g + 1)
        descs[("t", g)].wait()
        acc = compute(g, acc)
        if g + 2 < _NCHUNK:
            fire_idx(g + 2)
    acc_v[...] = acc
    pltpu.async_copy(acc_v, out_hbm.at[wid], so).wait()


@functools.lru_cache(maxsize=1)
def _build_sc_gather():
    return pl.kernel(
        _sc_body,
        out_type=jax.ShapeDtypeStruct((_NW, _L), jnp.float32),
        mesh=plsc.VectorSubcoreMesh(
            core_axis_name="c", subcore_axis_name="s",
            num_cores=_NC, num_subcores=_NS),
        scratch_types=[
            pltpu.VMEM((N_ITEMS,), jnp.float32),
            pltpu.VMEM((6 * N_ITEMS,), jnp.float32),
            pltpu.VMEM_SHARED((N_PERSONS,), jnp.float32),
            pltpu.VMEM((_L,), jnp.float32),
            pltpu.VMEM((3 * _C,), jnp.int32),
            pltpu.VMEM((3 * _C,), jnp.int32),
            pltpu.VMEM((_C,), jnp.int32),
            pltpu.VMEM((_C,), jnp.int32),
            pltpu.VMEM((_C,), jnp.float32),
            pltpu.VMEM((_C,), jnp.float32),
            pltpu.SemaphoreType.DMA,
            pltpu.SemaphoreType.DMA,
            pltpu.SemaphoreType.DMA,
            pltpu.SemaphoreType.DMA,
            pltpu.SemaphoreType.DMA,
        ],
        compiler_params=pltpu.CompilerParams(needs_layout_passes=False),
    )


def _finish_body(ps_ref, aw_ref, c0_ref, c1_ref, c2_ref, c3_ref,
                 t_ref, bpm_ref, bps_ref, lvl_ref, out_ref):
    ll = jnp.sum(ps_ref[...])

    aw = aw_ref[...]
    lp = jnp.sum(-0.5 * aw * aw) - 0.5 * _LOG2PI * N_ITEMS

    lvl = lvl_ref[...]
    masks = [(lvl == l).astype(jnp.float32) for l in range(N_LEVELS)]
    cols = (c0_ref[...], c1_ref[...], c2_ref[...], c3_ref[...])
    for c in range(N_GRADES - 1):
        mw = jnp.zeros((80, 125), jnp.float32)
        ivw = jnp.zeros((80, 125), jnp.float32)
        lsw = jnp.zeros((80, 125), jnp.float32)
        for l in range(N_LEVELS):
            m11 = bpm_ref[l:l + 1, c:c + 1]
            s11 = _softplus(bps_ref[l:l + 1, c:c + 1])
            mw = mw + masks[l] * m11
            ivw = ivw + masks[l] * (1.0 / s11)
            lsw = lsw + masks[l] * jnp.log(s11)
        z = (cols[c] - mw) * ivw
        lp += jnp.sum(-0.5 * z * z - lsw)
    lp -= 0.5 * _LOG2PI * (N_ITEMS * (N_GRADES - 1))

    t = t_ref[...]
    lp += jnp.sum(-0.5 * t * t) - 0.5 * _LOG2PI * N_PERSONS
    bpm = bpm_ref[...]
    bps = _softplus(bps_ref[...])
    lp += jnp.sum(-0.5 * bpm * bpm) - 0.5 * _LOG2PI * (N_LEVELS * (N_GRADES - 1))
    lp += jnp.sum(-2.0 * jnp.log(bps) - 1.0 / bps)

    out_ref[...] = jnp.reshape(-(ll + lp), (1, 1))


def _finish(psums, aw, c0, c1, c2, c3, t2d, bpm, bps_, lvlw):
    return pl.pallas_call(
        _finish_body,
        out_shape=jax.ShapeDtypeStruct((1, 1), jnp.float32),
    )(psums, aw, c0, c1, c2, c3, t2d, bpm, bps_, lvlw)


def kernel(a_, b_base_, b_diff_, t, b_prior_mean, b_prior_std_, indices,
           level_index):
    aw, c0, c1, c2, c3 = _make_tables(
        a_.reshape(80, 125), b_base_.reshape(80, 125),
        b_diff_[:, 0].reshape(80, 125), b_diff_[:, 1].reshape(80, 125),
        b_diff_[:, 2].reshape(80, 125))

    blk = jnp.full((N_ITEMS,), 1000.0, jnp.float32)
    b6t = jnp.concatenate([-blk, c0.reshape(-1), c1.reshape(-1),
                           c2.reshape(-1), c3.reshape(-1), blk])

    psums = _build_sc_gather()(aw.reshape(N_ITEMS), b6t, t,
                               indices.reshape(3 * N_RESP).astype(jnp.int32))

    out = _finish(psums, aw, c0, c1, c2, c3, t.reshape(800, 125),
                  b_prior_mean, b_prior_std_,
                  level_index.reshape(80, 125).astype(jnp.int32))
    return out.reshape(())


# speedup vs baseline: 24.0694x; 24.0694x over previous
import functools
import math

import jax
import jax.numpy as jnp
from jax import lax
from jax.experimental import pallas as pl
from jax.experimental.pallas import tpu as pltpu
from jax.experimental.pallas import tpu_sc as plsc

N_ITEMS = 10000
N_PERSONS = 100000
N_GRADES = 5
N_RESP = 1000000
N_LEVELS = 10

_NC, _NS, _L = 2, 16, 16
_NW = _NC * _NS
_NPAD = 1 << 20
_W = _NPAD // _NW
_C = 4096
_NCHUNK = _W // _C

_LOG2PI = math.log(2.0 * math.pi)
_LN2 = math.log(2.0)


def _softplus(x):
    return jnp.maximum(x, 0.0) + jnp.log(1.0 + jnp.exp(-jnp.abs(x)))


def _tables_body(a_ref, bb_ref, d0_ref, d1_ref, d2_ref,
                 aw_ref, c0_ref, c1_ref, c2_ref, c3_ref):
    aw_ref[...] = _softplus(a_ref[...])
    c0 = bb_ref[...]
    c1 = c0 + _softplus(d0_ref[...])
    c2 = c1 + _softplus(d1_ref[...])
    c3 = c2 + _softplus(d2_ref[...])
    c0_ref[...] = c0
    c1_ref[...] = c1
    c2_ref[...] = c2
    c3_ref[...] = c3


def _make_tables(aw, bbw, d0, d1, d2):
    w = jax.ShapeDtypeStruct((80, 125), jnp.float32)
    return pl.pallas_call(
        _tables_body, out_shape=[w, w, w, w, w],
    )(aw, bbw, d0, d1, d2)


def _ln16(p):
    bits = plsc.bitcast(p, jnp.int32)
    e = lax.shift_right_logical(bits, 23) - 127
    m = plsc.bitcast((bits & 0x007FFFFF) | 0x3F800000, jnp.float32)
    s = (m - 1.0) / (m + 1.0)
    s2 = s * s
    atanh = s * (1.0 + s2 * (1.0 / 3.0 + s2 * (0.2 + s2 * (1.0 / 7.0))))
    return e.astype(jnp.float32) * _LN2 + 2.0 * atanh


def _sc_body(a_hbm, b6_hbm, t_hbm, item_hbm, person_hbm, resp_hbm, out_hbm,
             a_v, b6_v, t_sh, acc_v,
             i0, i1, pn0, pn1, r0, r1, t0, t1,
             si0, si1, st0, st1, so):
    cid = lax.axis_index("c")
    sid = lax.axis_index("s")
    wid = sid * _NC + cid
    base = wid * _W

    @pl.when(sid == 0)
    def _():
        pltpu.sync_copy(t_hbm, t_sh)

    pltpu.sync_copy(a_hbm, a_v)
    pltpu.sync_copy(b6_hbm, b6_v)
    plsc.subcore_barrier()

    ib = (i0, i1)
    pb = (pn0, pn1)
    rb = (r0, r1)
    tb = (t0, t1)
    isem = (si0, si1)
    tsem = (st0, st1)
    descs = {}

    def fire_idx(g):
        off = base + g * _C
        b = g % 2
        descs[("i", g)] = [
            pltpu.async_copy(item_hbm.at[pl.ds(off, _C)], ib[b], isem[b]),
            pltpu.async_copy(person_hbm.at[pl.ds(off, _C)], pb[b], isem[b]),
            pltpu.async_copy(resp_hbm.at[pl.ds(off, _C)], rb[b], isem[b]),
        ]

    def fire_t(g):
        b = g % 2
        descs[("t", g)] = pltpu.async_copy(t_sh.at[pb[b]], tb[b], tsem[b])

    lane = lax.iota(jnp.int32, _L)

    def compute(g, acc_in):
        b = g % 2
        off = base + g * _C

        @plsc.parallel_loop(0, _C, step=_L, unroll=4, carry=acc_in)
        def acc_out(i, acc):
            s = pl.ds(i, _L)
            it = ib[b][s]
            rs = rb[b][s]
            fhi = it * 6 + rs
            a16 = plsc.load_gather(a_v, [it])
            bl = plsc.load_gather(b6_v, [fhi - 1])
            bu = plsc.load_gather(b6_v, [fhi])
            tt = tb[b][s]
            zl = jnp.maximum(a16 * (tt - bl), -30.0)
            zu = jnp.maximum(a16 * (tt - bu), -30.0)
            x = jnp.exp(-zl)
            y = jnp.exp(-zu)
            p = (y - x) / ((1.0 + x) * (1.0 + y))
            lnp = _ln16(jnp.maximum(p, 1e-37))
            pos = off + i + lane
            return acc + jnp.where(pos < N_RESP, lnp, 0.0)

        return acc_out

    fire_idx(0)
    for d in descs[("i", 0)]:
        d.wait()
    fire_t(0)
    if _NCHUNK > 1:
        fire_idx(1)
    acc = jnp.zeros((_L,), jnp.float32)
    for g in range(_NCHUNK):
        descs[("t", g)].wait()
        if g + 1 < _NCHUNK:
            for d in descs[("i", g + 1)]:
                d.wait()
            fire_t(g + 1)
        acc = compute(g, acc)
        if g + 2 < _NCHUNK:
            fire_idx(g + 2)
    acc_v[...] = acc
    pltpu.async_copy(acc_v, out_hbm.at[wid], so).wait()


@functools.lru_cache(maxsize=1)
def _build_sc_gather():
    return pl.kernel(
        _sc_body,
        out_type=jax.ShapeDtypeStruct((_NW, _L), jnp.float32),
        mesh=plsc.VectorSubcoreMesh(
            core_axis_name="c", subcore_axis_name="s",
            num_cores=_NC, num_subcores=_NS),
        scratch_types=[
            pltpu.VMEM((N_ITEMS,), jnp.float32),
            pltpu.VMEM((6 * N_ITEMS,), jnp.float32),
            pltpu.VMEM_SHARED((N_PERSONS,), jnp.float32),
            pltpu.VMEM((_L,), jnp.float32),
            pltpu.VMEM((_C,), jnp.int32),
            pltpu.VMEM((_C,), jnp.int32),
            pltpu.VMEM((_C,), jnp.int32),
            pltpu.VMEM((_C,), jnp.int32),
            pltpu.VMEM((_C,), jnp.int32),
            pltpu.VMEM((_C,), jnp.int32),
            pltpu.VMEM((_C,), jnp.float32),
            pltpu.VMEM((_C,), jnp.float32),
            pltpu.SemaphoreType.DMA,
            pltpu.SemaphoreType.DMA,
            pltpu.SemaphoreType.DMA,
            pltpu.SemaphoreType.DMA,
            pltpu.SemaphoreType.DMA,
        ],
        compiler_params=pltpu.CompilerParams(needs_layout_passes=False),
    )


def _finish_body(ps_ref, aw_ref, c0_ref, c1_ref, c2_ref, c3_ref,
                 t_ref, bpm_ref, bps_ref, lvl_ref, out_ref):
    ll = jnp.sum(ps_ref[...])

    aw = aw_ref[...]
    lp = jnp.sum(-0.5 * aw * aw) - 0.5 * _LOG2PI * N_ITEMS

    lvl = lvl_ref[...]
    masks = [(lvl == l).astype(jnp.float32) for l in range(N_LEVELS)]
    cols = (c0_ref[...], c1_ref[...], c2_ref[...], c3_ref[...])
    for c in range(N_GRADES - 1):
        mw = jnp.zeros((80, 125), jnp.float32)
        ivw = jnp.zeros((80, 125), jnp.float32)
        lsw = jnp.zeros((80, 125), jnp.float32)
        for l in range(N_LEVELS):
            m11 = bpm_ref[l:l + 1, c:c + 1]
            s11 = _softplus(bps_ref[l:l + 1, c:c + 1])
            mw = mw + masks[l] * m11
            ivw = ivw + masks[l] * (1.0 / s11)
            lsw = lsw + masks[l] * jnp.log(s11)
        z = (cols[c] - mw) * ivw
        lp += jnp.sum(-0.5 * z * z - lsw)
    lp -= 0.5 * _LOG2PI * (N_ITEMS * (N_GRADES - 1))

    t = t_ref[...]
    lp += jnp.sum(-0.5 * t * t) - 0.5 * _LOG2PI * N_PERSONS
    bpm = bpm_ref[...]
    bps = _softplus(bps_ref[...])
    lp += jnp.sum(-0.5 * bpm * bpm) - 0.5 * _LOG2PI * (N_LEVELS * (N_GRADES - 1))
    lp += jnp.sum(-2.0 * jnp.log(bps) - 1.0 / bps)

    out_ref[...] = jnp.reshape(-(ll + lp), (1, 1))


def _finish(psums, aw, c0, c1, c2, c3, t2d, bpm, bps_, lvlw):
    return pl.pallas_call(
        _finish_body,
        out_shape=jax.ShapeDtypeStruct((1, 1), jnp.float32),
    )(psums, aw, c0, c1, c2, c3, t2d, bpm, bps_, lvlw)


def kernel(a_, b_base_, b_diff_, t, b_prior_mean, b_prior_std_, indices,
           level_index):
    aw, c0, c1, c2, c3 = _make_tables(
        a_.reshape(80, 125), b_base_.reshape(80, 125),
        b_diff_[:, 0].reshape(80, 125), b_diff_[:, 1].reshape(80, 125),
        b_diff_[:, 2].reshape(80, 125))

    blk = jnp.full((N_ITEMS,), 1000.0, jnp.float32)
    b6i = jnp.stack([-blk, c0.reshape(-1), c1.reshape(-1), c2.reshape(-1),
                     c3.reshape(-1), blk], axis=1).reshape(6 * N_ITEMS)

    npad = _NPAD - N_RESP
    zpad = jnp.zeros((npad,), jnp.int32)
    item = jnp.concatenate([indices[:, 0].astype(jnp.int32), zpad])
    person = jnp.concatenate([indices[:, 1].astype(jnp.int32), zpad])
    resp = jnp.concatenate([indices[:, 2].astype(jnp.int32),
                            jnp.ones((npad,), jnp.int32)])

    psums = _build_sc_gather()(aw.reshape(N_ITEMS), b6i,
                               t, item, person, resp)

    out = _finish(psums, aw, c0, c1, c2, c3,
                  t.reshape(800, 125), b_prior_mean, b_prior_std_,
                  level_index.reshape(80, 125).astype(jnp.int32))
    return out.reshape(())
